# Initial kernel scaffold; baseline (speedup 1.0000x reference)
import jax
import jax.numpy as jnp
from jax.experimental import pallas as pl


def kernel(x, edge_index, W1, b1, W2, b2, W3, b3):
    raise NotImplementedError("write your pallas kernel here")


# speedup vs baseline: 3.7816x; 3.7816x over previous
import functools

import jax
import jax.numpy as jnp
from jax import lax
from jax.experimental import pallas as pl
from jax.experimental.pallas import tpu as pltpu
from jax.experimental.pallas import tpu_sc as plsc

N_NODES = 10000
N_EDGES = 320000
FDIM = 128

NC = 2
NS = 16
NW = NC * NS
K = 128
NPAD = 10240
SLAB = NPAD // NS
EPW = -(-N_EDGES // NW)
CHUNKS = -(-EPW // K)
EPW_PAD = CHUNKS * K
E_PAD = EPW_PAD * NW

R = 256
GRID = NPAD // R


def _sc_agg_body(table, ei, z2, z1, agg_out, deg_out,
                 s_idx, d_idx, rows, ones, acc, deg_acc, sem):
    c = lax.axis_index("c")
    s = lax.axis_index("s")
    slab = s * SLAB
    pltpu.sync_copy(z2.at[pl.ds(slab, SLAB)], acc.at[pl.ds(slab, SLAB)])
    pltpu.sync_copy(z1.at[pl.ds(slab, SLAB)], deg_acc.at[pl.ds(slab, SLAB)])
    for i in range(K // 16):
        ones[pl.ds(i * 16, 16)] = jnp.full((16,), 1.0, jnp.float32)
    plsc.subcore_barrier()

    base = (c * NS + s) * EPW_PAD

    def step(j, carry):
        off = base + j * K
        pltpu.sync_copy(ei.at[pl.ds(off, K)], s_idx)
        pltpu.sync_copy(ei.at[pl.ds(E_PAD + off, K)], d_idx)
        pltpu.async_copy(table.at[s_idx], rows, sem).wait()
        pltpu.sync_copy(rows, acc.at[d_idx], add=True)
        pltpu.sync_copy(ones, deg_acc.at[d_idx], add=True)
        return carry

    lax.fori_loop(0, CHUNKS, step, 0)
    plsc.subcore_barrier()

    out_off = c * NPAD + slab
    pltpu.sync_copy(acc.at[pl.ds(slab, SLAB)], agg_out.at[pl.ds(out_off, SLAB)])
    pltpu.sync_copy(deg_acc.at[pl.ds(slab, SLAB)], deg_out.at[pl.ds(out_off, SLAB)])


def _sc_aggregate(table, ei_flat, z2, z1):
    mesh = plsc.VectorSubcoreMesh(core_axis_name="c", subcore_axis_name="s")
    f = pl.kernel(
        _sc_agg_body,
        out_type=(
            jax.ShapeDtypeStruct((NC * NPAD, FDIM), jnp.float32),
            jax.ShapeDtypeStruct((NC * NPAD,), jnp.float32),
        ),
        mesh=mesh,
        scratch_types=[
            pltpu.VMEM((K,), jnp.int32),
            pltpu.VMEM((K,), jnp.int32),
            pltpu.VMEM((K, FDIM), jnp.float32),
            pltpu.VMEM((K,), jnp.float32),
            pltpu.VMEM_SHARED((NPAD, FDIM), jnp.float32),
            pltpu.VMEM_SHARED((NPAD,), jnp.float32),
            pltpu.SemaphoreType.DMA,
        ],
    )
    return f(table, ei_flat, z2, z1)


def _mm_relu_body(a0, a1, d0, d1, w, b, o):
    d = jnp.maximum(d0[...] + d1[...], 1.0)
    h = (a0[...] + a1[...]) / d
    y = jnp.dot(h, w[...], preferred_element_type=jnp.float32) + b[...]
    o[...] = jnp.maximum(y, 0.0)


def _mm_fused_body(a0, a1, d0, d1, w2, b2, w3, b3, o):
    d = jnp.maximum(d0[...] + d1[...], 1.0)
    h = (a0[...] + a1[...]) / d
    h = jnp.maximum(jnp.dot(h, w2[...], preferred_element_type=jnp.float32) + b2[...], 0.0)
    o[...] = jnp.dot(h, w3[...], preferred_element_type=jnp.float32) + b3[...]


_A0 = pl.BlockSpec((R, FDIM), lambda i: (i, 0))
_A1 = pl.BlockSpec((R, FDIM), lambda i: (i + GRID, 0))
_D0 = pl.BlockSpec((R, 1), lambda i: (i, 0))
_D1 = pl.BlockSpec((R, 1), lambda i: (i + GRID, 0))
_W = pl.BlockSpec((FDIM, FDIM), lambda i: (0, 0))
_B = pl.BlockSpec((1, FDIM), lambda i: (0, 0))
_O = pl.BlockSpec((R, FDIM), lambda i: (i, 0))


def _mm_relu(agg, deg2, w, b):
    return pl.pallas_call(
        _mm_relu_body,
        grid=(GRID,),
        in_specs=[_A0, _A1, _D0, _D1, _W, _B],
        out_specs=_O,
        out_shape=jax.ShapeDtypeStruct((NPAD, FDIM), jnp.float32),
    )(agg, agg, deg2, deg2, w, b)


def _mm_fused(agg, deg2, w2, b2, w3, b3):
    return pl.pallas_call(
        _mm_fused_body,
        grid=(GRID,),
        in_specs=[_A0, _A1, _D0, _D1, _W, _B, _W, _B],
        out_specs=_O,
        out_shape=jax.ShapeDtypeStruct((NPAD, FDIM), jnp.float32),
    )(agg, agg, deg2, deg2, w2, b2, w3, b3)


def kernel(x, edge_index, W1, b1, W2, b2, W3, b3):
    src = jnp.pad(edge_index[0].astype(jnp.int32), (0, E_PAD - N_EDGES),
                  constant_values=0)
    dst = jnp.pad(edge_index[1].astype(jnp.int32), (0, E_PAD - N_EDGES),
                  constant_values=N_NODES)
    ei_flat = jnp.concatenate([src, dst])
    z2 = jnp.zeros((NPAD, FDIM), jnp.float32)
    z1 = jnp.zeros((NPAD,), jnp.float32)

    agg1, deg = _sc_aggregate(x, ei_flat, z2, z1)
    deg2 = deg.reshape(NC * NPAD, 1)
    h1 = _mm_relu(agg1, deg2, W1, b1[None, :])
    agg2, _ = _sc_aggregate(h1, ei_flat, z2, z1)
    out = _mm_fused(agg2, deg2, W2, b2[None, :], W3, b3[None, :])
    return out[:N_NODES]
